# baseline (speedup 1.0000x reference)
import functools

import jax
import jax.numpy as jnp
from jax import lax
from jax.experimental import pallas as pl
from jax.experimental.pallas import tpu as pltpu
from jax.experimental.pallas import tpu_sc as plsc

_IDX_CHUNK = 128
_TILE = 128


@functools.lru_cache(maxsize=None)
def _build(V, B, fs, w, h):
    info = plsc.get_sparse_core_info()
    NC, NS = info.num_cores, info.num_subcores
    NW = NC * NS
    bpw = B // NW
    nchunk = bpw // _IDX_CHUNK
    mesh = plsc.VectorSubcoreMesh(core_axis_name="c", subcore_axis_name="s")
    scale = (fs, fs, w, h)

    @functools.partial(
        pl.kernel,
        mesh=mesh,
        compiler_params=pltpu.CompilerParams(needs_layout_passes=False),
        out_type=tuple(
            jax.ShapeDtypeStruct((B,), jnp.float32) for _ in range(5)),
        scratch_types=[
            pltpu.VMEM((nchunk, _IDX_CHUNK), jnp.int32),
            pltpu.VMEM((nchunk, _IDX_CHUNK), jnp.int32),
            pltpu.VMEM((_IDX_CHUNK // 2, 4, _TILE), jnp.float32),
            pltpu.VMEM((_IDX_CHUNK // 2, 4, _TILE), jnp.float32),
        ] + [pltpu.VMEM((bpw,), jnp.float32) for _ in range(5)] + [
            pltpu.SemaphoreType.DMA,
            pltpu.SemaphoreType.DMA,
            pltpu.SemaphoreType.DMA,
        ],
    )
    def k(idx_hbm, tab3_hbm, dist_hbm,
          fx_hbm, fy_hbm, x0_hbm, y0_hbm, dout_hbm,
          idx_v, tidx_v, blk0, blk1, g0, g1, g2, g3, g4,
          sem_a, sem_b, sem_d):
        wid = lax.axis_index("s") * NC + lax.axis_index("c")
        pltpu.sync_copy(idx_hbm.at[pl.ds(wid * nchunk, nchunk)], idx_v)
        gbufs = (g0, g1, g2, g3)
        blks = (blk0, blk1)
        sems = (sem_a, sem_b)

        dcopies = [
            pltpu.async_copy(
                dist_hbm.at[idx_v.at[j]],
                g4.at[pl.ds(j * _IDX_CHUNK, _IDX_CHUNK)], sem_d)
            for j in range(nchunk)
        ]

        def tile_step(t, carry):
            j = t // 8
            sl = pl.ds(16 * (t % 8), 16)
            tidx_v[j, sl] = lax.shift_right_logical(idx_v[j, sl], 7)
            return carry

        lax.fori_loop(0, nchunk * 8, tile_step, 0)

        iota = lax.iota(jnp.int32, 16)
        half = _IDX_CHUNK // 2
        nh = 2 * nchunk

        def fire(h):
            j, p = h // 2, h % 2
            return pltpu.async_copy(
                tab3_hbm.at[tidx_v.at[j, pl.ds(p * half, half)]],
                blks[h % 2], sems[h % 2])

        cps = {0: fire(0)}
        for h in range(nh):
            cps[h].wait()
            if h + 1 < nh:
                cps[h + 1] = fire(h + 1)
            j, p = h // 2, h % 2
            blk = blks[h % 2]

            def extract_step(g, carry, j=j, p=p, blk=blk):
                jv = 16 * g + iota
                rv = idx_v[j, pl.ds(p * half + 16 * g, 16)]
                iv = rv & 127
                for ci in range(4):
                    v = plsc.load_gather(blk, [jv, iota * 0 + ci, iv])
                    gbufs[ci][
                        pl.ds(j * _IDX_CHUNK + p * half + 16 * g, 16)] = (
                        v * jnp.float32(scale[ci]))
                return carry

            lax.fori_loop(0, half // 16, extract_step, 0)

        for c in dcopies:
            c.wait()
        dsts = (fx_hbm, fy_hbm, x0_hbm, y0_hbm, dout_hbm)
        allbufs = (g0, g1, g2, g3, g4)
        for ci, dst_hbm in enumerate(dsts):
            pltpu.sync_copy(allbufs[ci], dst_hbm.at[pl.ds(wid * bpw, bpw)])

    return k


def kernel(input, video_idx, intrinsics_factors, distortion):
    H, W = input.shape[1], input.shape[2]
    fs = 0.5 * (H + W)
    V = intrinsics_factors.shape[0]
    B = video_idx.shape[0]
    k = _build(V, B, float(fs), float(W), float(H))
    idx32 = video_idx.astype(jnp.int32)
    nt = (V + _TILE - 1) // _TILE
    padded = jnp.pad(intrinsics_factors, ((0, nt * _TILE - V), (0, 0)))
    view3 = padded.reshape(nt, _TILE, 4).transpose(0, 2, 1)
    fx, fy, x0, y0, dist = k(
        idx32.reshape(-1, _IDX_CHUNK),
        view3,
        distortion,
    )
    zero = jnp.zeros_like(fx)
    one = jnp.ones_like(fx)
    row0 = jnp.stack([fx, zero, x0], axis=-1)
    row1 = jnp.stack([zero, fy, y0], axis=-1)
    row2 = jnp.stack([zero, zero, one], axis=-1)
    int_mat = jnp.stack([row0, row1, row2], axis=1)
    return int_mat, dist.reshape(B, 1, 1)


# speedup vs baseline: 1.0431x; 1.0431x over previous
import functools

import jax
import jax.numpy as jnp
from jax import lax
from jax.experimental import pallas as pl
from jax.experimental.pallas import tpu as pltpu
from jax.experimental.pallas import tpu_sc as plsc

_IDX_CHUNK = 128
_TILE = 128


@functools.lru_cache(maxsize=None)
def _build(V, B, fs, w, h):
    info = plsc.get_sparse_core_info()
    NC, NS = info.num_cores, info.num_subcores
    NW = NC * NS
    bpw = B // NW
    nchunk = bpw // _IDX_CHUNK
    mesh = plsc.VectorSubcoreMesh(core_axis_name="c", subcore_axis_name="s")
    scale = (fs, fs, w, h)

    @functools.partial(
        pl.kernel,
        mesh=mesh,
        compiler_params=pltpu.CompilerParams(needs_layout_passes=False),
        out_type=tuple(
            jax.ShapeDtypeStruct((B,), jnp.float32) for _ in range(5)),
        scratch_types=[
            pltpu.VMEM((nchunk, _IDX_CHUNK), jnp.int32),
            pltpu.VMEM((nchunk, _IDX_CHUNK), jnp.int32),
            pltpu.VMEM((_IDX_CHUNK, 4, _TILE), jnp.float32),
        ] + [pltpu.VMEM((bpw,), jnp.float32) for _ in range(5)] + [
            pltpu.SemaphoreType.DMA,
            pltpu.SemaphoreType.DMA,
        ],
    )
    def k(idx_hbm, tab3_hbm, dist_hbm,
          fx_hbm, fy_hbm, x0_hbm, y0_hbm, dout_hbm,
          idx_v, tidx_v, blk_v, g0, g1, g2, g3, g4, sem_c, sem_d):
        wid = lax.axis_index("s") * NC + lax.axis_index("c")
        pltpu.sync_copy(idx_hbm.at[pl.ds(wid * nchunk, nchunk)], idx_v)
        gbufs = (g0, g1, g2, g3)

        dcopies = [
            pltpu.async_copy(
                dist_hbm.at[idx_v.at[j]],
                g4.at[pl.ds(j * _IDX_CHUNK, _IDX_CHUNK)], sem_d)
            for j in range(nchunk)
        ]

        def tile_step(t, carry):
            j = t // 8
            sl = pl.ds(16 * (t % 8), 16)
            tidx_v[j, sl] = lax.shift_right_logical(idx_v[j, sl], 7)
            return carry

        lax.fori_loop(0, nchunk * 8, tile_step, 0)

        iota = lax.iota(jnp.int32, 16)

        for j in range(nchunk):
            pltpu.async_copy(
                tab3_hbm.at[tidx_v.at[j]], blk_v, sem_c).wait()

            def extract_step(g, carry, j=j):
                jv = 16 * g + iota
                rv = idx_v[j, pl.ds(16 * g, 16)]
                iv = rv & 127
                for ci in range(4):
                    v = plsc.load_gather(blk_v, [jv, iota * 0 + ci, iv])
                    gbufs[ci][pl.ds(j * _IDX_CHUNK + 16 * g, 16)] = (
                        v * jnp.float32(scale[ci]))
                return carry

            lax.fori_loop(0, _IDX_CHUNK // 16, extract_step, 0)

        for c in dcopies:
            c.wait()
        dsts = (fx_hbm, fy_hbm, x0_hbm, y0_hbm, dout_hbm)
        allbufs = (g0, g1, g2, g3, g4)
        for ci, dst_hbm in enumerate(dsts):
            pltpu.sync_copy(allbufs[ci], dst_hbm.at[pl.ds(wid * bpw, bpw)])

    return k


def kernel(input, video_idx, intrinsics_factors, distortion):
    H, W = input.shape[1], input.shape[2]
    fs = 0.5 * (H + W)
    V = intrinsics_factors.shape[0]
    B = video_idx.shape[0]
    k = _build(V, B, float(fs), float(W), float(H))
    idx32 = video_idx.astype(jnp.int32)
    nt = (V + _TILE - 1) // _TILE
    padded = jnp.pad(intrinsics_factors, ((0, nt * _TILE - V), (0, 0)))
    view3 = padded.reshape(nt, _TILE, 4).transpose(0, 2, 1)
    fx, fy, x0, y0, dist = k(
        idx32.reshape(-1, _IDX_CHUNK),
        view3,
        distortion,
    )
    zero = jnp.zeros_like(fx)
    one = jnp.ones_like(fx)
    row0 = jnp.stack([fx, zero, x0], axis=-1)
    row1 = jnp.stack([zero, fy, y0], axis=-1)
    row2 = jnp.stack([zero, zero, one], axis=-1)
    int_mat = jnp.stack([row0, row1, row2], axis=1)
    return int_mat, dist.reshape(B, 1, 1)
